# baseline (speedup 1.0000x reference)
import functools

import jax
import jax.numpy as jnp
from jax.experimental import pallas as pl
from jax.experimental.pallas import tpu as pltpu


def _stage_body(adj_a_ref, adj_b_ref, adj_c_ref, feat_ref, h_ref, W_ref,
                b_ref, Wm_ref, bg_ref, out_ref):
    feat = feat_ref[...]
    t = jnp.concatenate(
        (
            jnp.dot(adj_a_ref[...], feat, preferred_element_type=jnp.float32),
            jnp.dot(adj_b_ref[...], feat, preferred_element_type=jnp.float32),
            jnp.dot(adj_c_ref[...], feat, preferred_element_type=jnp.float32),
        ),
        axis=0,
    )
    gcn = jnp.maximum(
        jnp.dot(t, W_ref[...], preferred_element_type=jnp.float32) + b_ref[...],
        0.0,
    )
    h = h_ref[...]
    gate = jax.nn.sigmoid(
        jnp.dot(h, Wm_ref[...], preferred_element_type=jnp.float32) + bg_ref[...]
    )
    out_ref[...] = gate * gcn + (1.0 - gate) * h


def _stage(adj, feat, W, b, Wm, bg, tm):
    m, k = adj.shape
    d = feat.shape[1]
    grid = (pl.cdiv(m, 3 * tm),)
    return pl.pallas_call(
        _stage_body,
        grid=grid,
        in_specs=[
            pl.BlockSpec((tm, k), lambda i: (3 * i, 0)),
            pl.BlockSpec((tm, k), lambda i: (3 * i + 1, 0)),
            pl.BlockSpec((tm, k), lambda i: (3 * i + 2, 0)),
            pl.BlockSpec((k, d), lambda i: (0, 0)),
            pl.BlockSpec((3 * tm, d), lambda i: (i, 0)),
            pl.BlockSpec((d, d), lambda i: (0, 0)),
            pl.BlockSpec((1, d), lambda i: (0, 0)),
            pl.BlockSpec((d, d), lambda i: (0, 0)),
            pl.BlockSpec((1, d), lambda i: (0, 0)),
        ],
        out_specs=pl.BlockSpec((3 * tm, d), lambda i: (i, 0)),
        out_shape=jax.ShapeDtypeStruct((m, d), jnp.float32),
        compiler_params=pltpu.CompilerParams(
            dimension_semantics=("arbitrary",),
        ),
    )(adj, adj, adj, feat, feat, W, b, Wm, bg)


@functools.partial(jax.jit, static_argnames=())
def kernel(e_x, r_x, prim_adj, rela_adj, W1, b1, Wm, bg, W2, b2):
    b1r = b1.reshape(1, -1)
    b2r = b2.reshape(1, -1)
    bgr = bg.reshape(1, -1)
    x = _stage(prim_adj, e_x, W1, b1r, Wm, bgr, tm=176)
    feat2 = jnp.concatenate((x, r_x), axis=0)
    x2 = _stage(rela_adj, feat2, W2, b2r, Wm, bgr, tm=176)
    return x2


# speedup vs baseline: 1.0347x; 1.0347x over previous
import jax
import jax.numpy as jnp
from jax.experimental import pallas as pl
from jax.experimental.pallas import tpu as pltpu


def _stage_body(adj_a_ref, adj_b_ref, feat_ref, W_ref, b_ref, Wm_ref, bg_ref,
                out_ref):
    i = pl.program_id(0)
    tm = adj_a_ref.shape[0]
    feat = feat_ref[...]
    t = jnp.concatenate(
        (
            jnp.dot(adj_a_ref[...], feat, preferred_element_type=jnp.float32),
            jnp.dot(adj_b_ref[...], feat, preferred_element_type=jnp.float32),
        ),
        axis=0,
    )
    gcn = jnp.maximum(
        jnp.dot(t, W_ref[...], preferred_element_type=jnp.float32) + b_ref[...],
        0.0,
    )
    h = feat_ref[pl.ds(i * 2 * tm, 2 * tm), :]
    gate = jax.nn.sigmoid(
        jnp.dot(h, Wm_ref[...], preferred_element_type=jnp.float32) + bg_ref[...]
    )
    out_ref[...] = gate * gcn + (1.0 - gate) * h


def _stage(adj, feat, W, b, Wm, bg, tm):
    m, k = adj.shape
    d = feat.shape[1]
    grid = (m // (2 * tm),)
    return pl.pallas_call(
        _stage_body,
        grid=grid,
        in_specs=[
            pl.BlockSpec((tm, k), lambda i: (2 * i, 0)),
            pl.BlockSpec((tm, k), lambda i: (2 * i + 1, 0)),
            pl.BlockSpec((k, d), lambda i: (0, 0)),
            pl.BlockSpec((d, d), lambda i: (0, 0)),
            pl.BlockSpec((1, d), lambda i: (0, 0)),
            pl.BlockSpec((d, d), lambda i: (0, 0)),
            pl.BlockSpec((1, d), lambda i: (0, 0)),
        ],
        out_specs=pl.BlockSpec((2 * tm, d), lambda i: (i, 0)),
        out_shape=jax.ShapeDtypeStruct((m, d), jnp.float32),
        compiler_params=pltpu.CompilerParams(
            dimension_semantics=("arbitrary",),
        ),
    )(adj, adj, feat, W, b, Wm, bg)


def kernel(e_x, r_x, prim_adj, rela_adj, W1, b1, Wm, bg, W2, b2):
    b1r = b1.reshape(1, -1)
    b2r = b2.reshape(1, -1)
    bgr = bg.reshape(1, -1)
    x = _stage(prim_adj, e_x, W1, b1r, Wm, bgr, tm=200)
    feat2 = jnp.concatenate((x, r_x), axis=0)
    x2 = _stage(rela_adj, feat2, W2, b2r, Wm, bgr, tm=200)
    return x2
